# baseline (speedup 1.0000x reference)
import jax
import jax.numpy as jnp
from jax import lax
from jax.experimental import pallas as pl
from jax.experimental.pallas import tpu as pltpu
from jax.experimental.pallas import tpu_sc as plsc

_VOCAB = 1_000_000
_D = 64
_N = 4096
_U = 200
_NC = 2
_NS = 16
_NW = _NC * _NS
_SEQ_PER_W = _N // _NW
_L = 16
_KV = _D // _L


def _sc_decoder(y_hbm, table_hbm, w0_hbm, w1_hbm, out_hbm,
                idx_v, rows_v, out_v, w0_v, w1_v, sem):
    wid = lax.axis_index("s") * _NC + lax.axis_index("c")
    pltpu.sync_copy(w0_hbm, w0_v)
    pltpu.sync_copy(w1_hbm, w1_v)
    w0r = [w0_v[pl.ds(_L * k, _L)] for k in range(_KV)]
    w1r = [w1_v[pl.ds(_L * k, _L)] for k in range(_KV)]
    zero = jnp.zeros((_L,), jnp.float32)

    def seq_body(j, carry):
        base = (wid * _SEQ_PER_W + j) * _U
        pltpu.sync_copy(y_hbm.at[pl.ds(base, _U)], idx_v)
        c1 = pltpu.async_copy(table_hbm.at[idx_v.at[pl.ds(0, 128)]],
                              rows_v.at[pl.ds(0, 128)], sem)
        c2 = pltpu.async_copy(table_hbm.at[idx_v.at[pl.ds(128, _U - 128)]],
                              rows_v.at[pl.ds(128, _U - 128)], sem)
        c1.wait()
        c2.wait()

        def row_body(i, prev):
            cur = []
            for k in range(_KV):
                c = rows_v[i, pl.ds(_L * k, _L)]
                out_v[i, pl.ds(_L * k, _L)] = jnp.maximum(
                    c * w1r[k] + prev[k] * w0r[k], 0.0)
                cur.append(c)
            return tuple(cur)

        lax.fori_loop(0, _U, row_body, (zero,) * _KV)
        pltpu.sync_copy(out_v, out_hbm.at[pl.ds(base, _U)])
        return carry

    lax.fori_loop(0, _SEQ_PER_W, seq_body, 0)


def kernel(y, emb_weight, conv_weight):
    assert y.shape == (_N, _U) and emb_weight.shape == (_VOCAB, _D)
    y_idx = jnp.clip(y, 0, _VOCAB - 1).astype(jnp.int32).reshape(_N * _U)
    w0 = conv_weight[:, 0, 0]
    w1 = conv_weight[:, 0, 1]
    mesh = plsc.VectorSubcoreMesh(core_axis_name="c", subcore_axis_name="s")
    f = pl.kernel(
        _sc_decoder,
        mesh=mesh,
        compiler_params=pltpu.CompilerParams(use_tc_tiling_on_sc=False),
        out_type=jax.ShapeDtypeStruct((_N * _U, _D), jnp.float32),
        scratch_types=[
            pltpu.VMEM((_U,), jnp.int32),
            pltpu.VMEM((_U, _D), jnp.float32),
            pltpu.VMEM((_U, _D), jnp.float32),
            pltpu.VMEM((_D,), jnp.float32),
            pltpu.VMEM((_D,), jnp.float32),
            pltpu.SemaphoreType.DMA,
        ],
    )
    out = f(y_idx, emb_weight, w0, w1)
    return out.reshape(_N, _U, _D)


# speedup vs baseline: 1.1832x; 1.1832x over previous
import jax
import jax.numpy as jnp
from jax import lax
from jax.experimental import pallas as pl
from jax.experimental.pallas import tpu as pltpu
from jax.experimental.pallas import tpu_sc as plsc

_VOCAB = 1_000_000
_D = 64
_N = 4096
_U = 200
_NC = 2
_NS = 16
_NW = _NC * _NS
_SEQ_PER_W = _N // _NW
_L = 16
_KV = _D // _L
_C1 = 128
_C2 = _U - _C1


def _sc_decoder(y_hbm, table_hbm, w0_hbm, w1_hbm, out_hbm,
                idx_v, rows0, rows1, out0, out1, w0_v, w1_v,
                gsem0, gsem1, ssem0, ssem1):
    wid = lax.axis_index("s") * _NC + lax.axis_index("c")
    wbase = wid * _SEQ_PER_W * _U
    pltpu.sync_copy(w0_hbm, w0_v)
    pltpu.sync_copy(w1_hbm, w1_v)
    pltpu.sync_copy(y_hbm.at[pl.ds(wbase, _SEQ_PER_W * _U)], idx_v)
    w0r = [w0_v[pl.ds(_L * k, _L)] for k in range(_KV)]
    w1r = [w1_v[pl.ds(_L * k, _L)] for k in range(_KV)]
    zero = jnp.zeros((_L,), jnp.float32)
    rows = (rows0, rows1)
    outs = (out0, out1)
    gsems = (gsem0, gsem1)
    ssems = (ssem0, ssem1)

    def fire_gather(j, p):
        off = j * _U
        pltpu.async_copy(table_hbm.at[idx_v.at[pl.ds(off, _C1)]],
                         rows[p].at[pl.ds(0, _C1)], gsems[p])
        pltpu.async_copy(table_hbm.at[idx_v.at[pl.ds(off + _C1, _C2)]],
                         rows[p].at[pl.ds(_C1, _C2)], gsems[p])

    def wait_gather(p):
        pltpu.make_async_copy(table_hbm.at[idx_v.at[pl.ds(0, _C1)]],
                              rows[p].at[pl.ds(0, _C1)], gsems[p]).wait()
        pltpu.make_async_copy(table_hbm.at[idx_v.at[pl.ds(_C1, _C2)]],
                              rows[p].at[pl.ds(_C1, _C2)], gsems[p]).wait()

    def compute(p):
        def row_body(i, prev):
            cur = []
            for k in range(_KV):
                c = rows[p][i, pl.ds(_L * k, _L)]
                outs[p][i, pl.ds(_L * k, _L)] = jnp.maximum(
                    c * w1r[k] + prev[k] * w0r[k], 0.0)
                cur.append(c)
            return tuple(cur)
        lax.fori_loop(0, _U, row_body, (zero,) * _KV)

    def fire_store(j, p):
        pltpu.async_copy(outs[p], out_hbm.at[pl.ds(wbase + j * _U, _U)],
                         ssems[p])

    def wait_store(p):
        pltpu.make_async_copy(outs[p], out_hbm.at[pl.ds(wbase, _U)],
                              ssems[p]).wait()

    fire_gather(0, 0)
    fire_gather(1, 1)

    def step(j, p):
        wait_gather(p)

        @pl.when(j >= 2)
        def _():
            wait_store(p)

        compute(p)
        fire_store(j, p)

        @pl.when(j + 2 < _SEQ_PER_W)
        def _():
            fire_gather(j + 2, p)

    def pair_body(jj, carry):
        step(2 * jj, 0)
        step(2 * jj + 1, 1)
        return carry

    lax.fori_loop(0, _SEQ_PER_W // 2, pair_body, 0)
    wait_store(0)
    wait_store(1)


def kernel(y, emb_weight, conv_weight):
    assert y.shape == (_N, _U) and emb_weight.shape == (_VOCAB, _D)
    y_idx = jnp.clip(y, 0, _VOCAB - 1).astype(jnp.int32).reshape(_N * _U)
    w0 = conv_weight[:, 0, 0]
    w1 = conv_weight[:, 0, 1]
    mesh = plsc.VectorSubcoreMesh(core_axis_name="c", subcore_axis_name="s")
    f = pl.kernel(
        _sc_decoder,
        mesh=mesh,
        compiler_params=pltpu.CompilerParams(use_tc_tiling_on_sc=False),
        out_type=jax.ShapeDtypeStruct((_N * _U, _D), jnp.float32),
        scratch_types=[
            pltpu.VMEM((_SEQ_PER_W * _U,), jnp.int32),
            pltpu.VMEM((_U, _D), jnp.float32),
            pltpu.VMEM((_U, _D), jnp.float32),
            pltpu.VMEM((_U, _D), jnp.float32),
            pltpu.VMEM((_U, _D), jnp.float32),
            pltpu.VMEM((_D,), jnp.float32),
            pltpu.VMEM((_D,), jnp.float32),
            pltpu.SemaphoreType.DMA,
            pltpu.SemaphoreType.DMA,
            pltpu.SemaphoreType.DMA,
            pltpu.SemaphoreType.DMA,
        ],
    )
    out = f(y_idx, emb_weight, w0, w1)
    return out.reshape(_N, _U, _D)
